# baseline (speedup 1.0000x reference)
import functools

import jax
import jax.numpy as jnp
from jax import lax
from jax.experimental import pallas as pl
from jax.experimental.pallas import tpu as pltpu
from jax.experimental.pallas import tpu_sc as plsc

B = 4096
HIDDEN = 768
MAX_POS = 77
NC, NS, L = 2, 16, 16
NW = NC * NS
SEQ_PER_W = B // NW
VPR = HIDDEN // L
GRP = 16
P = 40
OFF1 = MAX_POS - P

_mesh = plsc.VectorSubcoreMesh(core_axis_name="c", subcore_axis_name="s")


@functools.partial(
    pl.kernel,
    out_type=jax.ShapeDtypeStruct((B, 2, P, HIDDEN), jnp.float32),
    mesh=_mesh,
    scratch_types=[
        pltpu.VMEM((P,), jnp.int32),
        pltpu.VMEM((P,), jnp.int32),
        pltpu.VMEM((P, HIDDEN), jnp.float32),
        pltpu.VMEM((P, HIDDEN), jnp.float32),
        pltpu.VMEM((P, HIDDEN), jnp.float32),
        pltpu.VMEM((P, HIDDEN), jnp.float32),
        pltpu.SemaphoreType.DMA,
        pltpu.SemaphoreType.DMA,
    ],
)
def _emb_kernel(ids0_hbm, ids1_hbm, tok_hbm, pos0_hbm, pos1_hbm, out_hbm,
                idx0, idx1, pos0_v, pos1_v, buf0, buf1, sg0, sg1):
  wid = lax.axis_index("s") * NC + lax.axis_index("c")
  b_base = wid * SEQ_PER_W

  pltpu.sync_copy(pos0_hbm, pos0_v)
  pltpu.sync_copy(pos1_hbm, pos1_v)

  def seq_body(s, carry):
    pltpu.sync_copy(ids0_hbm.at[b_base + s], idx0)
    pltpu.sync_copy(ids1_hbm.at[b_base + s], idx1)
    c0 = pltpu.async_copy(tok_hbm.at[idx0], buf0, sg0)
    c1 = pltpu.async_copy(tok_hbm.at[idx1], buf1, sg1)
    c0.wait()
    c1.wait()

    def add_row0(r, carry2):
      for g in range(VPR // GRP):
        tv = [buf0[r, pl.ds((g * GRP + k) * L, L)] for k in range(GRP)]
        pv = [pos0_v[r, pl.ds((g * GRP + k) * L, L)] for k in range(GRP)]
        for k in range(GRP):
          buf0[r, pl.ds((g * GRP + k) * L, L)] = tv[k] + pv[k]
      return carry2

    def add_row1(r, carry2):
      for g in range(VPR // GRP):
        tv = [buf1[r, pl.ds((g * GRP + k) * L, L)] for k in range(GRP)]
        pv = [pos1_v[r, pl.ds((g * GRP + k) * L, L)] for k in range(GRP)]
        for k in range(GRP):
          buf1[r, pl.ds((g * GRP + k) * L, L)] = tv[k] + pv[k]
      return carry2

    lax.fori_loop(0, P, add_row0, 0)
    lax.fori_loop(0, P, add_row1, 0)
    pltpu.sync_copy(buf0, out_hbm.at[b_base + s, 0])
    pltpu.sync_copy(buf1, out_hbm.at[b_base + s, 1])
    return carry

  lax.fori_loop(0, SEQ_PER_W, seq_body, 0)


def kernel(input_ids, token_table, pos_table):
  ids = input_ids.astype(jnp.int32)
  ids0 = ids[:, :P]
  ids1 = ids[:, OFF1:]
  out4 = _emb_kernel(ids0, ids1, token_table,
                     pos_table[:P], pos_table[OFF1:])
  return jnp.concatenate([out4[:, 0], out4[:, 1, P - OFF1:]], axis=1)


# speedup vs baseline: 1.4208x; 1.4208x over previous
import functools

import jax
import jax.numpy as jnp
from jax import lax
from jax.experimental import pallas as pl
from jax.experimental.pallas import tpu as pltpu
from jax.experimental.pallas import tpu_sc as plsc

B = 4096
HIDDEN = 768
MAX_POS = 77
PADDED = 80
NC, NS, L = 2, 16, 16
NW = NC * NS
SEQ_PER_W = B // NW
VPR = HIDDEN // L
GRP = 16

_mesh = plsc.VectorSubcoreMesh(core_axis_name="c", subcore_axis_name="s")


@functools.partial(
    pl.kernel,
    out_type=jax.ShapeDtypeStruct((B, PADDED, HIDDEN), jnp.float32),
    mesh=_mesh,
    scratch_types=[
        pltpu.VMEM((PADDED,), jnp.int32),
        pltpu.VMEM((PADDED, HIDDEN), jnp.float32),
        pltpu.VMEM((PADDED, HIDDEN), jnp.float32),
        pltpu.SemaphoreType.DMA,
    ],
)
def _emb_kernel(ids_hbm, tok_hbm, pos_hbm, out_hbm,
                idx_v, pos_v, buf, sem):
  wid = lax.axis_index("s") * NC + lax.axis_index("c")
  b_base = wid * SEQ_PER_W

  pltpu.sync_copy(pos_hbm, pos_v)

  def seq_body(s, carry):
    pltpu.sync_copy(ids_hbm.at[b_base + s], idx_v)
    pltpu.async_copy(tok_hbm.at[idx_v], buf, sem).wait()

    def add_row(r, carry2):
      for g in range(VPR // GRP):
        tv = [buf[r, pl.ds((g * GRP + k) * L, L)] for k in range(GRP)]
        pv = [pos_v[r, pl.ds((g * GRP + k) * L, L)] for k in range(GRP)]
        for k in range(GRP):
          buf[r, pl.ds((g * GRP + k) * L, L)] = tv[k] + pv[k]
      return carry2

    lax.fori_loop(0, PADDED, add_row, 0)
    pltpu.sync_copy(buf, out_hbm.at[b_base + s])
    return carry

  lax.fori_loop(0, SEQ_PER_W, seq_body, 0)


def kernel(input_ids, token_table, pos_table):
  ids = input_ids.astype(jnp.int32)
  ids80 = jnp.pad(ids, ((0, 0), (0, PADDED - MAX_POS)))
  pos80 = jnp.pad(pos_table, ((0, PADDED - MAX_POS), (0, 0)))
  out = _emb_kernel(ids80, token_table, pos80)
  return out[:, :MAX_POS, :]
